# baseline (speedup 1.0000x reference)
import numpy as np
import jax
import jax.numpy as jnp
from jax import lax
from jax.experimental import pallas as pl
from jax.experimental.pallas import tpu as pltpu
from jax.experimental.pallas import tpu_sc as plsc

_B = 4096
_F = 256
_M = 128
_NPAIR = _M * (_M + 1) // 2
_OUT = _M + _NPAIR
_NW = 32
_RPW = _B // _NW
_II, _JJ = np.triu_indices(_M)


def _body(x_hbm, a_hbm, b_hbm, m_hbm, out_hbm, xblk, av_v, bv_v, mi_v, orow):
    wid = lax.axis_index("s") * 2 + lax.axis_index("c")
    base = wid * _RPW
    pltpu.sync_copy(x_hbm.at[pl.ds(base * _F, _RPW * _F)], xblk)
    pltpu.sync_copy(a_hbm, av_v)
    pltpu.sync_copy(b_hbm, bv_v)
    pltpu.sync_copy(m_hbm, mi_v)

    @pl.loop(0, _RPW)
    def _row(r):
        roff = jnp.full((16,), r * _F, jnp.int32)

        @pl.loop(0, _M // 16, unroll=8)
        def _copy(c):
            mi = mi_v[pl.ds(c * 16, 16)]
            orow[pl.ds(c * 16, 16)] = plsc.load_gather(xblk, [roff + mi])

        @pl.loop(0, _NPAIR // 16, unroll=8)
        def _de(t):
            a = av_v[pl.ds(t * 16, 16)]
            b = bv_v[pl.ds(t * 16, 16)]
            va = plsc.load_gather(xblk, [roff + a])
            vb = plsc.load_gather(xblk, [roff + b])
            orow[pl.ds(_M + t * 16, 16)] = va * vb

        pltpu.sync_copy(orow, out_hbm.at[pl.ds((base + r) * _OUT, _OUT)])


def _mask_de(xflat, a2, b2, midx):
    f = pl.kernel(
        _body,
        out_type=jax.ShapeDtypeStruct((_B * _OUT,), jnp.float32),
        mesh=plsc.VectorSubcoreMesh(core_axis_name="c", subcore_axis_name="s"),
        compiler_params=pltpu.CompilerParams(needs_layout_passes=False),
        scratch_types=[
            pltpu.VMEM((_RPW * _F,), jnp.float32),
            pltpu.VMEM((_NPAIR,), jnp.int32),
            pltpu.VMEM((_NPAIR,), jnp.int32),
            pltpu.VMEM((_M,), jnp.int32),
            pltpu.VMEM((_OUT,), jnp.float32),
        ],
    )
    return f(xflat, a2, b2, midx)


def kernel(x, mask):
    midx = jnp.argsort(~mask)[:_M].astype(jnp.int32)
    a2 = midx[_II]
    b2 = midx[_JJ]
    out = _mask_de(x.reshape(-1), a2, b2, midx)
    return out.reshape(_B, _OUT)


# speedup vs baseline: 3.6565x; 3.6565x over previous
import numpy as np
import jax
import jax.numpy as jnp
from jax import lax
from jax.experimental import pallas as pl
from jax.experimental.pallas import tpu as pltpu
from jax.experimental.pallas import tpu_sc as plsc

_B = 4096
_F = 256
_M = 128
_NPAIR = _M * (_M + 1) // 2
_OUT = _M + _NPAIR
_NW = 32
_RPW = _B // _NW
_L = 16
_XPAD = _M + _L
_OPAD = _OUT + _L


def _seg_off(i):
    return i * _M - i * (i - 1) // 2


def _compute_row(r, xblk, mi_v, xmb, buf):
    roffv = jnp.full((_L,), r * _F, jnp.int32)
    for c in range(_M // _L):
        mi = mi_v[pl.ds(c * _L, _L)]
        v = plsc.load_gather(xblk, [roffv + mi])
        xmb[pl.ds(c * _L, _L)] = v
        buf[pl.ds(c * _L, _L)] = v
    for i in range(_M):
        if i % _L == 0:
            svec = xmb[pl.ds(i, _L)]
        sv = jnp.full((_L,), svec[i % _L])
        base = _M + _seg_off(i)
        for k in range(-(-(_M - i) // _L)):
            v = xmb[pl.ds(i + _L * k, _L)]
            buf[pl.ds(base + _L * k, _L)] = sv * v


def _body(x_hbm, m_hbm, out_hbm, xblk, mi_v, xmb, bufa, bufb, sema, semb):
    wid = lax.axis_index("s") * 2 + lax.axis_index("c")
    base = wid * _RPW
    pltpu.sync_copy(x_hbm.at[pl.ds(base * _F, _RPW * _F)], xblk)
    pltpu.sync_copy(m_hbm, mi_v)

    def _dst(r):
        return out_hbm.at[pl.ds((base + r) * _OUT, _OUT)]

    @pl.loop(0, _RPW, step=2)
    def _rows(r0):
        @pl.when(r0 > 0)
        def _():
            pltpu.make_async_copy(bufa.at[pl.ds(0, _OUT)], _dst(r0 - 2), sema).wait()

        _compute_row(r0, xblk, mi_v, xmb, bufa)
        pltpu.async_copy(bufa.at[pl.ds(0, _OUT)], _dst(r0), sema)

        @pl.when(r0 > 0)
        def _():
            pltpu.make_async_copy(bufb.at[pl.ds(0, _OUT)], _dst(r0 - 1), semb).wait()

        _compute_row(r0 + 1, xblk, mi_v, xmb, bufb)
        pltpu.async_copy(bufb.at[pl.ds(0, _OUT)], _dst(r0 + 1), semb)

    pltpu.make_async_copy(bufa.at[pl.ds(0, _OUT)], _dst(_RPW - 2), sema).wait()
    pltpu.make_async_copy(bufb.at[pl.ds(0, _OUT)], _dst(_RPW - 1), semb).wait()


def _mask_de(xflat, midx):
    f = pl.kernel(
        _body,
        out_type=jax.ShapeDtypeStruct((_B * _OUT,), jnp.float32),
        mesh=plsc.VectorSubcoreMesh(core_axis_name="c", subcore_axis_name="s"),
        compiler_params=pltpu.CompilerParams(needs_layout_passes=False),
        scratch_types=[
            pltpu.VMEM((_RPW * _F,), jnp.float32),
            pltpu.VMEM((_M,), jnp.int32),
            pltpu.VMEM((_XPAD,), jnp.float32),
            pltpu.VMEM((_OPAD,), jnp.float32),
            pltpu.VMEM((_OPAD,), jnp.float32),
            pltpu.SemaphoreType.DMA,
            pltpu.SemaphoreType.DMA,
        ],
    )
    return f(xflat, midx)


def kernel(x, mask):
    midx = jnp.argsort(~mask)[:_M].astype(jnp.int32)
    out = _mask_de(x.reshape(-1), midx)
    return out.reshape(_B, _OUT)
